# baseline (speedup 1.0000x reference)
import functools

import jax
import jax.numpy as jnp
from jax.experimental import pallas as pl
from jax.experimental.pallas import tpu as pltpu

_C = 256
_MAXD = 32
_THR = 0.5
_ANCHOR = 224.0
_NEG = -1e30


def _rpn_kernel(feat_ref, w5_ref, b5_ref, chw_ref, msc_ref, *, W, stride):
    feat = feat_ref[0]
    lin = jnp.dot(w5_ref[...], feat, preferred_element_type=jnp.float32)
    lin = lin + b5_ref[...]
    obj = jax.nn.sigmoid(lin[0:1])
    d = jnp.tanh(lin[1:5])
    HW = feat.shape[1]
    idx = jax.lax.broadcasted_iota(jnp.int32, (1, HW), 1)
    gy = (idx // W).astype(jnp.float32) + 0.5
    gx = (idx % W).astype(jnp.float32) + 0.5
    base = _ANCHOR / stride
    cx = (gx + d[0:1]) * stride
    cy = (gy + d[1:2]) * stride
    bw = (base * jnp.exp(d[2:3])) * stride
    bh = (base * jnp.exp(d[3:4])) * stride
    chw_ref[0] = jnp.concatenate([cx, cy, bw, bh], axis=0)
    lv = jnp.clip(jnp.floor(3.0 + jnp.log2(jnp.sqrt(bw * bh) / 224.0)), 1.0, 4.0)
    msc_ref[0] = jnp.concatenate(
        [jnp.where(lv == 1.0, obj, _NEG),
         jnp.where(lv == 2.0, obj, _NEG),
         jnp.where(lv == 3.0, obj, _NEG)], axis=0)


def _nms_kernel(cx_ref, cy_ref, w_ref, h_ref, sc_ref,
                kcx_ref, kcy_ref, kw_ref, kh_ref, kval_ref):
    cx = cx_ref[...]
    cy = cy_ref[...]
    w = w_ref[...]
    h = h_ref[...]
    sc = sc_ref[...]
    R, N = sc.shape
    x1 = cx - w / 2
    y1 = cy - h / 2
    x2 = cx + w / 2
    y2 = cy + h / 2
    area = (x2 - x1) * (y2 - y1)
    iota = jax.lax.broadcasted_iota(jnp.int32, (R, N), 1)
    ci = jax.lax.broadcasted_iota(jnp.int32, (R, _MAXD), 1)
    zero = jnp.zeros((R, _MAXD), jnp.float32)
    acc_cx, acc_cy, acc_w, acc_h, acc_v = zero, zero, zero, zero, zero

    def pick(onehot, a):
        return jnp.sum(jnp.where(onehot, a, 0.0), axis=1, keepdims=True)

    for d in range(_MAXD):
        m = jnp.max(sc, axis=1, keepdims=True)
        val = (m > -1e29).astype(jnp.float32)
        idx = jnp.min(jnp.where(sc == m, iota, N), axis=1, keepdims=True)
        onehot = iota == idx
        bx1 = pick(onehot, x1)
        by1 = pick(onehot, y1)
        bx2 = pick(onehot, x2)
        by2 = pick(onehot, y2)
        ix1 = jnp.maximum(bx1, x1)
        iy1 = jnp.maximum(by1, y1)
        ix2 = jnp.minimum(bx2, x2)
        iy2 = jnp.minimum(by2, y2)
        inter = jnp.maximum(ix2 - ix1, 0.0) * jnp.maximum(iy2 - iy1, 0.0)
        a1 = (bx2 - bx1) * (by2 - by1)
        iou = inter / (a1 + area - inter + 1e-9)
        sc = jnp.where(iou > _THR, _NEG, sc)
        sc = jnp.where(onehot, _NEG, sc)
        hit = ci == d
        acc_cx = jnp.where(hit, pick(onehot, cx) * val, acc_cx)
        acc_cy = jnp.where(hit, pick(onehot, cy) * val, acc_cy)
        acc_w = jnp.where(hit, pick(onehot, w) * val, acc_w)
        acc_h = jnp.where(hit, pick(onehot, h) * val, acc_h)
        acc_v = jnp.where(hit, val, acc_v)
    kcx_ref[...] = acc_cx
    kcy_ref[...] = acc_cy
    kw_ref[...] = acc_w
    kh_ref[...] = acc_h
    kval_ref[...] = acc_v


def _roi_body(box_ref, val_ref, f3_ref, out_ref, row, b, H, W):
    bx = box_ref[row, 0]
    by = box_ref[row, 1]
    bw = box_ref[row, 2]
    bh = box_ref[row, 3]
    vf = val_ref[row, 0]
    rw = jnp.maximum(bw - bx, 1.0)
    rh = jnp.maximum(bh - by, 1.0)
    jj = jax.lax.broadcasted_iota(jnp.int32, (1, 14), 1).astype(jnp.float32)
    xs = bx + (jj + 0.5) * (rw / 14.0)
    xs = jnp.clip(xs, 0.0, W - 1.0)
    x0 = jnp.floor(xs)
    x1i = jnp.minimum(x0 + 1.0, W - 1.0)
    wx = xs - x0
    wiota = jax.lax.broadcasted_iota(jnp.int32, (W, 14), 0).astype(jnp.float32)
    rxt = (jnp.where(wiota == x0, 1.0 - wx, 0.0)
           + jnp.where(wiota == x1i, wx, 0.0))
    rh14 = rh / 14.0
    for i in range(14):
        ys = by + (i + 0.5) * rh14
        ys = jnp.clip(ys, 0.0, H - 1.0)
        yi = ys.astype(jnp.int32)
        wy = ys - yi.astype(jnp.float32)
        y1i = jnp.minimum(yi + 1, H - 1)
        a = f3_ref[b, pl.ds(yi * _C, _C), :]
        bb = f3_ref[b, pl.ds(y1i * _C, _C), :]
        t = a * ((1.0 - wy) * vf) + bb * (wy * vf)
        out_ref[0, :, 14 * i:14 * i + 14] = jnp.dot(
            t, rxt, preferred_element_type=jnp.float32)


def _roi_kernel(box_ref, val_ref, fa_ref, fb_ref, fc_ref, out_ref):
    nl = pl.program_id(0)
    b = pl.program_id(1)
    d = pl.program_id(2)
    row = b * 96 + nl * 32 + d

    @pl.when(nl == 0)
    def _():
        _roi_body(box_ref, val_ref, fa_ref, out_ref, row, b, 16, 16)

    @pl.when(nl == 1)
    def _():
        _roi_body(box_ref, val_ref, fb_ref, out_ref, row, b, 32, 32)

    @pl.when(nl == 2)
    def _():
        _roi_body(box_ref, val_ref, fc_ref, out_ref, row, b, 64, 64)


def kernel(p32, p16, p8, p4,
           Wobj32, bobj32, Wbox32, bbox32,
           Wobj16, bobj16, Wbox16, bbox16,
           Wobj8, bobj8, Wbox8, bbox8):
    del p4
    B = p32.shape[0]
    lv_feats = [p32, p16, p8]
    lv_params = [(Wobj32, bobj32, Wbox32, bbox32, 32.0, 16, 16),
                 (Wobj16, bobj16, Wbox16, bbox16, 16.0, 32, 32),
                 (Wobj8, bobj8, Wbox8, bbox8, 8.0, 64, 64)]
    chws, mscs = [], []
    for p, (Wo, bo, Wb, bb, stride, H, W) in zip(lv_feats, lv_params):
        HW = H * W
        feat = p.reshape(B, _C, HW)
        w5 = jnp.concatenate([Wo[None, :], Wb.T], axis=0)
        b5 = jnp.concatenate([bo[None], bb], axis=0)[:, None]
        chw, msc = pl.pallas_call(
            functools.partial(_rpn_kernel, W=W, stride=stride),
            grid=(B,),
            in_specs=[pl.BlockSpec((1, _C, HW), lambda b: (b, 0, 0)),
                      pl.BlockSpec((5, _C), lambda b: (0, 0)),
                      pl.BlockSpec((5, 1), lambda b: (0, 0))],
            out_specs=[pl.BlockSpec((1, 4, HW), lambda b: (b, 0, 0)),
                       pl.BlockSpec((1, 3, HW), lambda b: (b, 0, 0))],
            out_shape=[jax.ShapeDtypeStruct((B, 4, HW), jnp.float32),
                       jax.ShapeDtypeStruct((B, 3, HW), jnp.float32)],
        )(feat, w5, b5)
        chws.append(chw)
        mscs.append(msc)
    chw_all = jnp.concatenate(chws, axis=2)
    msc_all = jnp.concatenate(mscs, axis=2)
    N = chw_all.shape[2]
    rep = lambda a: jnp.repeat(a, 3, axis=0)
    sc6 = msc_all.reshape(B * 3, N)
    kcx, kcy, kw, kh, kval = pl.pallas_call(
        _nms_kernel,
        out_shape=[jax.ShapeDtypeStruct((B * 3, _MAXD), jnp.float32)] * 5,
    )(rep(chw_all[:, 0]), rep(chw_all[:, 1]),
      rep(chw_all[:, 2]), rep(chw_all[:, 3]), sc6)
    f3s = [p.transpose(0, 2, 1, 3).reshape(B, H * _C, W)
           for p, (_, _, _, _, _, H, W) in zip(lv_feats, lv_params)]
    boxes = jnp.stack([kcx, kcy, kw, kh], axis=-1).reshape(
        B, 3, _MAXD, 4).reshape(B * 3 * _MAXD, 4)
    vals = kval.reshape(B * 3 * _MAXD, 1)
    full = lambda shp: pl.BlockSpec(shp, lambda nl, b, d: (0,) * len(shp))
    out = pl.pallas_call(
        _roi_kernel,
        grid=(3, B, _MAXD),
        in_specs=[pl.BlockSpec(memory_space=pltpu.SMEM),
                  pl.BlockSpec(memory_space=pltpu.SMEM),
                  full(f3s[0].shape), full(f3s[1].shape), full(f3s[2].shape)],
        out_specs=pl.BlockSpec((1, _C, 196),
                               lambda nl, b, d: (b * 96 + nl * _MAXD + d, 0, 0)),
        out_shape=jax.ShapeDtypeStruct((B * 3 * _MAXD, _C, 196), jnp.float32),
        compiler_params=pltpu.CompilerParams(
            dimension_semantics=("parallel", "parallel", "arbitrary")),
    )(boxes, vals, *f3s)
    return out.reshape(B, 3 * _MAXD, _C, 14, 14)


# speedup vs baseline: 1.0580x; 1.0580x over previous
import functools

import jax
import jax.numpy as jnp
from jax.experimental import pallas as pl
from jax.experimental.pallas import tpu as pltpu

_C = 256
_MAXD = 32
_THR = 0.5
_ANCHOR = 224.0
_NEG = -1e30


def _rpn_kernel(feat_ref, w5_ref, b5_ref, chw_ref, msc_ref, *, W, stride):
    feat = feat_ref[0]
    lin = jnp.dot(w5_ref[...], feat, preferred_element_type=jnp.float32)
    lin = lin + b5_ref[...]
    obj = jax.nn.sigmoid(lin[0:1])
    d = jnp.tanh(lin[1:5])
    HW = feat.shape[1]
    idx = jax.lax.broadcasted_iota(jnp.int32, (1, HW), 1)
    gy = (idx // W).astype(jnp.float32) + 0.5
    gx = (idx % W).astype(jnp.float32) + 0.5
    base = _ANCHOR / stride
    cx = (gx + d[0:1]) * stride
    cy = (gy + d[1:2]) * stride
    bw = (base * jnp.exp(d[2:3])) * stride
    bh = (base * jnp.exp(d[3:4])) * stride
    chw_ref[0] = jnp.concatenate([cx, cy, bw, bh], axis=0)
    lv = jnp.clip(jnp.floor(3.0 + jnp.log2(jnp.sqrt(bw * bh) / 224.0)), 1.0, 4.0)
    msc_ref[0] = jnp.concatenate(
        [jnp.where(lv == 1.0, obj, _NEG),
         jnp.where(lv == 2.0, obj, _NEG),
         jnp.where(lv == 3.0, obj, _NEG)], axis=0)


def _nms_kernel(cx_ref, cy_ref, w_ref, h_ref, sc_ref,
                kcx_ref, kcy_ref, kw_ref, kh_ref, kval_ref):
    cx = cx_ref[...]
    cy = cy_ref[...]
    w = w_ref[...]
    h = h_ref[...]
    sc = sc_ref[...]
    R, N = sc.shape
    x1 = cx - w / 2
    y1 = cy - h / 2
    x2 = cx + w / 2
    y2 = cy + h / 2
    area = (x2 - x1) * (y2 - y1)
    iota = jax.lax.broadcasted_iota(jnp.int32, (R, N), 1)
    ci = jax.lax.broadcasted_iota(jnp.int32, (R, _MAXD), 1)
    zero = jnp.zeros((R, _MAXD), jnp.float32)
    acc_cx, acc_cy, acc_w, acc_h, acc_v = zero, zero, zero, zero, zero

    def pick(onehot, a):
        return jnp.sum(jnp.where(onehot, a, 0.0), axis=1, keepdims=True)

    for d in range(_MAXD):
        m = jnp.max(sc, axis=1, keepdims=True)
        val = (m > -1e29).astype(jnp.float32)
        idx = jnp.min(jnp.where(sc == m, iota, N), axis=1, keepdims=True)
        onehot = iota == idx
        bx1 = pick(onehot, x1)
        by1 = pick(onehot, y1)
        bx2 = pick(onehot, x2)
        by2 = pick(onehot, y2)
        ix1 = jnp.maximum(bx1, x1)
        iy1 = jnp.maximum(by1, y1)
        ix2 = jnp.minimum(bx2, x2)
        iy2 = jnp.minimum(by2, y2)
        inter = jnp.maximum(ix2 - ix1, 0.0) * jnp.maximum(iy2 - iy1, 0.0)
        a1 = (bx2 - bx1) * (by2 - by1)
        iou = inter / (a1 + area - inter + 1e-9)
        sc = jnp.where(iou > _THR, _NEG, sc)
        sc = jnp.where(onehot, _NEG, sc)
        hit = ci == d
        acc_cx = jnp.where(hit, pick(onehot, cx) * val, acc_cx)
        acc_cy = jnp.where(hit, pick(onehot, cy) * val, acc_cy)
        acc_w = jnp.where(hit, pick(onehot, w) * val, acc_w)
        acc_h = jnp.where(hit, pick(onehot, h) * val, acc_h)
        acc_v = jnp.where(hit, val, acc_v)
    kcx_ref[...] = acc_cx
    kcy_ref[...] = acc_cy
    kw_ref[...] = acc_w
    kh_ref[...] = acc_h
    kval_ref[...] = acc_v


def _roi_body(box_ref, val_ref, f3_ref, out_ref, row, b, H, W):
    bx = box_ref[row, 0]
    by = box_ref[row, 1]
    bw = box_ref[row, 2]
    bh = box_ref[row, 3]
    vf = val_ref[row, 0]
    rw = jnp.maximum(bw - bx, 1.0)
    rh = jnp.maximum(bh - by, 1.0)
    jj = jax.lax.broadcasted_iota(jnp.int32, (1, 14), 1).astype(jnp.float32)
    xs = bx + (jj + 0.5) * (rw / 14.0)
    xs = jnp.clip(xs, 0.0, W - 1.0)
    x0 = jnp.floor(xs)
    x1i = jnp.minimum(x0 + 1.0, W - 1.0)
    wx = xs - x0
    wiota = jax.lax.broadcasted_iota(jnp.int32, (W, 14), 0).astype(jnp.float32)
    rxt = (jnp.where(wiota == x0, 1.0 - wx, 0.0)
           + jnp.where(wiota == x1i, wx, 0.0))
    rh14 = rh / 14.0
    P = max(1, 128 // W)
    for i0 in range(0, 14, P):
        cnt = min(P, 14 - i0)
        slabs, rhs_rows = [], []
        for k in range(cnt):
            i = i0 + k
            ys = by + (i + 0.5) * rh14
            ys = jnp.clip(ys, 0.0, H - 1.0)
            yi = ys.astype(jnp.int32)
            wy = ys - yi.astype(jnp.float32)
            y1i = jnp.minimum(yi + 1, H - 1)
            a = f3_ref[b, pl.ds(yi * _C, _C), :]
            bb = f3_ref[b, pl.ds(y1i * _C, _C), :]
            slabs.append(a * ((1.0 - wy) * vf) + bb * (wy * vf))
            pieces = []
            if k:
                pieces.append(jnp.zeros((W, 14 * k), jnp.float32))
            pieces.append(rxt)
            if cnt - 1 - k:
                pieces.append(jnp.zeros((W, 14 * (cnt - 1 - k)), jnp.float32))
            rhs_rows.append(pieces[0] if len(pieces) == 1
                            else jnp.concatenate(pieces, axis=1))
        lhs = slabs[0] if cnt == 1 else jnp.concatenate(slabs, axis=1)
        rhs = rhs_rows[0] if cnt == 1 else jnp.concatenate(rhs_rows, axis=0)
        out_ref[0, :, 14 * i0:14 * (i0 + cnt)] = jnp.dot(
            lhs, rhs, preferred_element_type=jnp.float32)


def _roi_kernel(box_ref, val_ref, fa_ref, fb_ref, fc_ref, out_ref):
    nl = pl.program_id(0)
    b = pl.program_id(1)
    d = pl.program_id(2)
    row = b * 96 + nl * 32 + d

    @pl.when(nl == 0)
    def _():
        _roi_body(box_ref, val_ref, fa_ref, out_ref, row, b, 16, 16)

    @pl.when(nl == 1)
    def _():
        _roi_body(box_ref, val_ref, fb_ref, out_ref, row, b, 32, 32)

    @pl.when(nl == 2)
    def _():
        _roi_body(box_ref, val_ref, fc_ref, out_ref, row, b, 64, 64)


def kernel(p32, p16, p8, p4,
           Wobj32, bobj32, Wbox32, bbox32,
           Wobj16, bobj16, Wbox16, bbox16,
           Wobj8, bobj8, Wbox8, bbox8):
    del p4
    B = p32.shape[0]
    lv_feats = [p32, p16, p8]
    lv_params = [(Wobj32, bobj32, Wbox32, bbox32, 32.0, 16, 16),
                 (Wobj16, bobj16, Wbox16, bbox16, 16.0, 32, 32),
                 (Wobj8, bobj8, Wbox8, bbox8, 8.0, 64, 64)]
    chws, mscs = [], []
    for p, (Wo, bo, Wb, bb, stride, H, W) in zip(lv_feats, lv_params):
        HW = H * W
        feat = p.reshape(B, _C, HW)
        w5 = jnp.concatenate([Wo[None, :], Wb.T], axis=0)
        b5 = jnp.concatenate([bo[None], bb], axis=0)[:, None]
        chw, msc = pl.pallas_call(
            functools.partial(_rpn_kernel, W=W, stride=stride),
            grid=(B,),
            in_specs=[pl.BlockSpec((1, _C, HW), lambda b: (b, 0, 0)),
                      pl.BlockSpec((5, _C), lambda b: (0, 0)),
                      pl.BlockSpec((5, 1), lambda b: (0, 0))],
            out_specs=[pl.BlockSpec((1, 4, HW), lambda b: (b, 0, 0)),
                       pl.BlockSpec((1, 3, HW), lambda b: (b, 0, 0))],
            out_shape=[jax.ShapeDtypeStruct((B, 4, HW), jnp.float32),
                       jax.ShapeDtypeStruct((B, 3, HW), jnp.float32)],
        )(feat, w5, b5)
        chws.append(chw)
        mscs.append(msc)
    chw_all = jnp.concatenate(chws, axis=2)
    msc_all = jnp.concatenate(mscs, axis=2)
    N = chw_all.shape[2]
    rep = lambda a: jnp.repeat(a, 3, axis=0)
    sc6 = msc_all.reshape(B * 3, N)
    kcx, kcy, kw, kh, kval = pl.pallas_call(
        _nms_kernel,
        out_shape=[jax.ShapeDtypeStruct((B * 3, _MAXD), jnp.float32)] * 5,
    )(rep(chw_all[:, 0]), rep(chw_all[:, 1]),
      rep(chw_all[:, 2]), rep(chw_all[:, 3]), sc6)
    f3s = [p.transpose(0, 2, 1, 3).reshape(B, H * _C, W)
           for p, (_, _, _, _, _, H, W) in zip(lv_feats, lv_params)]
    boxes = jnp.stack([kcx, kcy, kw, kh], axis=-1).reshape(
        B, 3, _MAXD, 4).reshape(B * 3 * _MAXD, 4)
    vals = kval.reshape(B * 3 * _MAXD, 1)
    full = lambda shp: pl.BlockSpec(shp, lambda nl, b, d: (0,) * len(shp))
    out = pl.pallas_call(
        _roi_kernel,
        grid=(3, B, _MAXD),
        in_specs=[pl.BlockSpec(memory_space=pltpu.SMEM),
                  pl.BlockSpec(memory_space=pltpu.SMEM),
                  full(f3s[0].shape), full(f3s[1].shape), full(f3s[2].shape)],
        out_specs=pl.BlockSpec((1, _C, 196),
                               lambda nl, b, d: (b * 96 + nl * _MAXD + d, 0, 0)),
        out_shape=jax.ShapeDtypeStruct((B * 3 * _MAXD, _C, 196), jnp.float32),
        compiler_params=pltpu.CompilerParams(
            dimension_semantics=("parallel", "parallel", "arbitrary")),
    )(boxes, vals, *f3s)
    return out.reshape(B, 3 * _MAXD, _C, 14, 14)
